# baseline (speedup 1.0000x reference)
import functools

import jax
import jax.numpy as jnp
from jax import lax
from jax.experimental import pallas as pl
from jax.experimental.pallas import tpu as pltpu
from jax.experimental.pallas import tpu_sc as plsc

N_ATOMS = 100000
N_EDGES = 6400000
N_SPLINE = 10000
CUTOFF = 0.6

NC, NS = 2, 16
NW = NC * NS
EPT = N_EDGES // NW
ECHUNK = 80
NCHUNK = EPT // ECHUNK
NG = ECHUNK // 16

APT = 3136
N_ATOMS_PAD = NW * APT
ACHUNK = 448
NACHUNK = APT // ACHUNK
AG = ACHUNK // 16

TBL_LEN = 2 * N_SPLINE + 16

_mesh = plsc.VectorSubcoreMesh(core_axis_name="c", subcore_axis_name="s",
                               num_cores=NC, num_subcores=NS)
_cparams = pltpu.CompilerParams(use_tc_tiling_on_sc=False,
                                needs_layout_passes=False)


def _build_zone_tables(g):
    gb = lax.bitcast_convert_type(g * g, jnp.int32)
    base = jnp.maximum(gb - 48, 0)
    j = jnp.arange(96, dtype=jnp.int32)
    vals = lax.bitcast_convert_type(base[:, None] + j[None, :], jnp.float32)
    pred = jnp.sqrt(vals) >= g[:, None]
    ft = jnp.argmax(pred, axis=1).astype(jnp.int32)
    bb = base + ft
    take = jnp.take_along_axis(
        pred, jnp.minimum(ft[:, None] + j[None, :32], 95), axis=1)
    m = jnp.sum(take.astype(jnp.int32) << j[None, :32], axis=1)
    return bb, m


def _approx_sqrt(d2):
    bits = lax.bitcast_convert_type(d2, jnp.int32)
    y = lax.bitcast_convert_type(0x5F3759DF - (bits >> 1), jnp.float32)
    for _ in range(3):
        y = y * (1.5 - 0.5 * d2 * y * y)
    return d2 * y


def _minimage(d):
    return jnp.where(d > 0.5, d - 1.0, jnp.where(d < -0.5, d + 1.0, d))


def _edge_body(row_hbm, col_hbm, xs_hbm, y_hbm, z_hbm,
               de0, de1, de2, de3, pa0, pa1, pa2, pa3, tblb_hbm, tblm_hbm,
               rho_out, pv_out,
               tblb_v, tblm_v, rho_v, row_v, col_v,
               xr_v, yr_v, zr_v, xc_v, yc_v, zc_v,
               didx_v, pidx_v,
               d0_v, d1_v, d2_v, d3_v, p0_v, p1_v, p2_v, p3_v,
               scr_v, acc_v, sem):
    wid = lax.axis_index("c") * NS + lax.axis_index("s")
    pltpu.sync_copy(tblb_hbm, tblb_v)
    pltpu.sync_copy(tblm_hbm, tblm_v)

    def zero_body(i, _):
        rho_v[pl.ds(i * 16, 16)] = jnp.zeros((16,), jnp.float32)
        return 0

    lax.fori_loop(0, N_ATOMS_PAD // 16, zero_body, 0)
    acc_v[...] = jnp.zeros((16,), jnp.float32)

    def chunk_body(k, _):
        base = wid * EPT + k * ECHUNK
        pltpu.sync_copy(row_hbm.at[pl.ds(base, ECHUNK)], row_v)
        pltpu.sync_copy(col_hbm.at[pl.ds(base, ECHUNK)], col_v)
        cps = [pltpu.async_copy(xs_hbm.at[row_v], xr_v, sem),
               pltpu.async_copy(y_hbm.at[row_v], yr_v, sem),
               pltpu.async_copy(z_hbm.at[row_v], zr_v, sem),
               pltpu.async_copy(xs_hbm.at[col_v], xc_v, sem),
               pltpu.async_copy(y_hbm.at[col_v], yc_v, sem),
               pltpu.async_copy(z_hbm.at[col_v], zc_v, sem)]
        for c in cps:
            c.wait()

        bcut = tblb_v[pl.ds(N_SPLINE, 16)]
        mcut = tblm_v[pl.ds(N_SPLINE, 16)]

        def pred(d2b, b, m):
            js = d2b - b
            sh = lax.shift_right_logical(m, js & 31) & 1
            return jnp.where(js < 0, False,
                             jnp.where(js > 31, True, sh == 1))

        for g in range(NG):
            sl = pl.ds(g * 16, 16)
            xsr = xr_v[sl]
            xsc = xc_v[sl]
            ti = lax.shift_right_logical(
                lax.bitcast_convert_type(xsr, jnp.int32), 31)
            tj = lax.shift_right_logical(
                lax.bitcast_convert_type(xsc, jnp.int32), 31)
            dx = _minimage(jnp.abs(xsr) - jnp.abs(xsc))
            dy = _minimage(yr_v[sl] - yc_v[sl])
            dz = _minimage(zr_v[sl] - zc_v[sl])
            d2 = ((dx * dx + dz * dz) + dy * dy) + 1e-12

            r = _approx_sqrt(d2)
            rb = jnp.minimum(jnp.maximum(r * 16665.0, 0.0), 9998.0)
            cand = rb.astype(jnp.int32)
            d2b = lax.bitcast_convert_type(d2, jnp.int32)
            b0 = plsc.load_gather(tblb_v, [cand])
            m0 = plsc.load_gather(tblm_v, [cand])
            b1 = plsc.load_gather(tblb_v, [cand + 1])
            m1 = plsc.load_gather(tblm_v, [cand + 1])
            cf = cand.astype(jnp.float32)
            cf = jnp.where(pred(d2b, b1, m1), cf + 1.0,
                           jnp.where(pred(d2b, b0, m0), cf, cf - 1.0))
            cf = jnp.minimum(jnp.maximum(cf, 0.0), 9998.0)
            idx = cf.astype(jnp.int32)

            mf = jnp.where(pred(d2b, bcut, mcut), 0.0, 1.0)

            didx_v[sl] = tj * (N_SPLINE - 1) + idx
            pidx_v[sl] = (ti * 2 + tj) * (N_SPLINE - 1) + idx
            scr_v[sl] = r - cf * 6.00060006000600e-05
            scr_v[pl.ds(ECHUNK + g * 16, 16)] = mf

        cps = [pltpu.async_copy(de0.at[didx_v], d0_v, sem),
               pltpu.async_copy(de1.at[didx_v], d1_v, sem),
               pltpu.async_copy(de2.at[didx_v], d2_v, sem),
               pltpu.async_copy(de3.at[didx_v], d3_v, sem),
               pltpu.async_copy(pa0.at[pidx_v], p0_v, sem),
               pltpu.async_copy(pa1.at[pidx_v], p1_v, sem),
               pltpu.async_copy(pa2.at[pidx_v], p2_v, sem),
               pltpu.async_copy(pa3.at[pidx_v], p3_v, sem)]
        for c in cps:
            c.wait()

        for g in range(NG):
            sl = pl.ds(g * 16, 16)
            dxs = scr_v[sl]
            mf = scr_v[pl.ds(ECHUNK + g * 16, 16)]
            dens = (d3_v[sl] + dxs * (d2_v[sl] + dxs * (d1_v[sl] + dxs * d0_v[sl]))) * mf
            plsc.addupdate_scatter(rho_v, [row_v[sl]], dens)
            pv = (p3_v[sl] + dxs * (p2_v[sl] + dxs * (p1_v[sl] + dxs * p0_v[sl]))) * mf
            acc_v[...] = acc_v[...] + pv
        return 0

    lax.fori_loop(0, NCHUNK, chunk_body, 0)

    pltpu.sync_copy(rho_v, rho_out.at[wid])
    pltpu.sync_copy(acc_v, pv_out.at[wid])


_EF32 = pltpu.VMEM((ECHUNK,), jnp.float32)
_EI32 = pltpu.VMEM((ECHUNK,), jnp.int32)

_edge_kernel = functools.partial(
    pl.kernel,
    out_type=(jax.ShapeDtypeStruct((NW, N_ATOMS_PAD), jnp.float32),
              jax.ShapeDtypeStruct((NW, 16), jnp.float32)),
    mesh=_mesh,
    compiler_params=_cparams,
    scratch_types=[
        pltpu.VMEM((N_SPLINE + 16,), jnp.int32),
        pltpu.VMEM((N_SPLINE + 16,), jnp.int32),
        pltpu.VMEM((N_ATOMS_PAD,), jnp.float32),
        _EI32, _EI32,
        _EF32, _EF32, _EF32, _EF32, _EF32, _EF32,
        _EI32, _EI32,
        _EF32, _EF32, _EF32, _EF32, _EF32, _EF32, _EF32, _EF32,
        pltpu.VMEM((2 * ECHUNK,), jnp.float32),
        pltpu.VMEM((16,), jnp.float32),
        pltpu.SemaphoreType.DMA,
    ],
)(_edge_body)


def _atom_body(rho_parts, types_hbm, g2_hbm, em0, em1, em2, em3,
               f_out,
               g2_v, rho32_v, types_v, eidx_v, edx_v,
               e0_v, e1_v, e2_v, e3_v, facc_v, sem):
    wid = lax.axis_index("c") * NS + lax.axis_index("s")
    pltpu.sync_copy(g2_hbm, g2_v)
    facc_v[...] = jnp.zeros((16,), jnp.float32)
    lanes = lax.iota(jnp.int32, 16)

    def chunk_body(j, _):
        abase = wid * APT + j * ACHUNK
        pltpu.sync_copy(rho_parts.at[:, pl.ds(abase, ACHUNK)], rho32_v)
        pltpu.sync_copy(types_hbm.at[pl.ds(abase, ACHUNK)], types_v)

        for g in range(AG):
            sl = pl.ds(g * 16, 16)
            rho = rho32_v[0, sl]
            for p in range(1, NW):
                rho = rho + rho32_v[p, sl]

            rc = jnp.minimum(jnp.maximum(rho, -8.0), 8.0)
            sf = jnp.minimum(jnp.maximum((rc + 8.0) * 624.9375, 0.0), 9998.0)
            ei = sf.astype(jnp.int32)
            gl = plsc.load_gather(g2_v, [ei])
            gr = plsc.load_gather(g2_v, [ei + 1])
            ef = ei.astype(jnp.float32)
            ef = jnp.where(rc >= gr, ef + 1.0, jnp.where(rc < gl, ef - 1.0, ef))
            ef = jnp.minimum(jnp.maximum(ef, 0.0), 9998.0)
            eidx = ef.astype(jnp.int32)
            gsel = plsc.load_gather(g2_v, [eidx])
            eidx_v[sl] = types_v[sl] * (N_SPLINE - 1) + eidx
            edx_v[sl] = rc - gsel

        cps = []
        for q in range(ACHUNK // 112):
            qs = pl.ds(q * 112, 112)
            iq = eidx_v.at[qs]
            cps += [pltpu.async_copy(em0.at[iq], e0_v.at[qs], sem),
                    pltpu.async_copy(em1.at[iq], e1_v.at[qs], sem),
                    pltpu.async_copy(em2.at[iq], e2_v.at[qs], sem),
                    pltpu.async_copy(em3.at[iq], e3_v.at[qs], sem)]
        for c in cps:
            c.wait()

        for g in range(AG):
            sl = pl.ds(g * 16, 16)
            edx = edx_v[sl]
            fv = e3_v[sl] + edx * (e2_v[sl] + edx * (e1_v[sl] + edx * e0_v[sl]))
            aid = abase + g * 16 + lanes
            valid = jnp.where(aid < N_ATOMS, 1.0, 0.0)
            facc_v[...] = facc_v[...] + fv * valid
        return 0

    lax.fori_loop(0, NACHUNK, chunk_body, 0)
    pltpu.sync_copy(facc_v, f_out.at[wid])


_AF32 = pltpu.VMEM((ACHUNK,), jnp.float32)

_atom_kernel = functools.partial(
    pl.kernel,
    out_type=jax.ShapeDtypeStruct((NW, 16), jnp.float32),
    mesh=_mesh,
    compiler_params=_cparams,
    scratch_types=[
        pltpu.VMEM((N_SPLINE,), jnp.float32),
        pltpu.VMEM((NW, ACHUNK), jnp.float32),
        pltpu.VMEM((ACHUNK,), jnp.int32),
        pltpu.VMEM((ACHUNK,), jnp.int32),
        _AF32,
        _AF32, _AF32, _AF32, _AF32,
        pltpu.VMEM((16,), jnp.float32),
        pltpu.SemaphoreType.DMA,
    ],
)(_atom_body)


def kernel(coords, edge_index, atom_types, spline_r_x, density_coeffs,
           embed_x, embed_coeffs, pair_coeffs):
    row = edge_index[0]
    col = edge_index[1]
    xs = jnp.where(atom_types == 1, -coords[:, 0], coords[:, 0])
    y = coords[:, 1]
    z = coords[:, 2]
    de = [density_coeffs[:, k_, :].reshape(-1) for k_ in range(4)]
    pa = [pair_coeffs[:, :, k_, :].reshape(-1) for k_ in range(4)]
    em = [embed_coeffs[:, k_, :].reshape(-1) for k_ in range(4)]

    gext = jnp.concatenate([spline_r_x, jnp.full((16,), CUTOFF, jnp.float32)])
    tbl_b, tbl_m = _build_zone_tables(gext)
    types_pad = jnp.concatenate(
        [atom_types, jnp.zeros((N_ATOMS_PAD - N_ATOMS,), jnp.int32)])

    rho_parts, pv_parts = _edge_kernel(
        row, col, xs, y, z, de[0], de[1], de[2], de[3],
        pa[0], pa[1], pa[2], pa[3], tbl_b, tbl_m)
    f_parts = _atom_kernel(rho_parts, types_pad, embed_x[0],
                           em[0], em[1], em[2], em[3])
    return jnp.sum(f_parts) + 0.5 * jnp.sum(pv_parts)


# speedup vs baseline: 1.7476x; 1.7476x over previous
import functools

import jax
import jax.numpy as jnp
from jax import lax
from jax.experimental import pallas as pl
from jax.experimental.pallas import tpu as pltpu
from jax.experimental.pallas import tpu_sc as plsc

N_ATOMS = 100000
N_EDGES = 6400000
N_SPLINE = 10000
CUTOFF = 0.6

NC, NS = 2, 16
NW = NC * NS
EPT = N_EDGES // NW
ECHUNK = 80
NCHUNK = EPT // ECHUNK
NG = ECHUNK // 16

APT = 3136
N_ATOMS_PAD = NW * APT
ACHUNK = 448
NACHUNK = APT // ACHUNK
AG = ACHUNK // 16

TBL_LEN = 2 * N_SPLINE + 16

_mesh = plsc.VectorSubcoreMesh(core_axis_name="c", subcore_axis_name="s",
                               num_cores=NC, num_subcores=NS)
_cparams = pltpu.CompilerParams(use_tc_tiling_on_sc=False,
                                needs_layout_passes=False)


def _build_zone_tables(g):
    gb = lax.bitcast_convert_type(g * g, jnp.int32)
    base = jnp.maximum(gb - 48, 0)
    j = jnp.arange(96, dtype=jnp.int32)
    vals = lax.bitcast_convert_type(base[:, None] + j[None, :], jnp.float32)
    pred = jnp.sqrt(vals) >= g[:, None]
    ft = jnp.argmax(pred, axis=1).astype(jnp.int32)
    bb = base + ft
    take = jnp.take_along_axis(
        pred, jnp.minimum(ft[:, None] + j[None, :32], 95), axis=1)
    m = jnp.sum(take.astype(jnp.int32) << j[None, :32], axis=1)
    return bb, m


def _approx_sqrt(d2):
    bits = lax.bitcast_convert_type(d2, jnp.int32)
    y = lax.bitcast_convert_type(0x5F3759DF - (bits >> 1), jnp.float32)
    for _ in range(3):
        y = y * (1.5 - 0.5 * d2 * y * y)
    return d2 * y


def _minimage(d):
    return jnp.where(d > 0.5, d - 1.0, jnp.where(d < -0.5, d + 1.0, d))


def _edge_body(row_hbm, col_hbm, xs_hbm, y_hbm, z_hbm,
               de0, de1, de2, de3, pa0, pa1, pa2, pa3, tblb_hbm, tblm_hbm,
               rho_out, pv_out,
               tblb_v, tblm_v, rho_v, row_v, col_v,
               xr_v, yr_v, zr_v, xc_v, yc_v, zc_v,
               didx_v, pidx_v,
               d0_v, d1_v, d2_v, d3_v, p0_v, p1_v, p2_v, p3_v,
               scr_v, acc_v, sem_c, sem_k, sem_rc):
    wid = lax.axis_index("c") * NS + lax.axis_index("s")
    pltpu.sync_copy(tblb_hbm, tblb_v)
    pltpu.sync_copy(tblm_hbm, tblm_v)

    def zero_body(i, _):
        rho_v[pl.ds(i * 16, 16)] = jnp.zeros((16,), jnp.float32)
        return 0

    lax.fori_loop(0, N_ATOMS_PAD // 16, zero_body, 0)
    acc_v[...] = jnp.zeros((16,), jnp.float32)

    def start_rc(k, p):
        base = wid * EPT + k * ECHUNK
        pltpu.async_copy(row_hbm.at[pl.ds(base, ECHUNK)], row_v.at[p], sem_rc)
        pltpu.async_copy(col_hbm.at[pl.ds(base, ECHUNK)], col_v.at[p], sem_rc)

    def wait_rc(p):
        pltpu.make_async_copy(row_hbm.at[pl.ds(0, ECHUNK)], row_v.at[p], sem_rc).wait()
        pltpu.make_async_copy(col_hbm.at[pl.ds(0, ECHUNK)], col_v.at[p], sem_rc).wait()

    def start_coords(p):
        pltpu.async_copy(xs_hbm.at[row_v.at[p]], xr_v.at[p], sem_c)
        pltpu.async_copy(y_hbm.at[row_v.at[p]], yr_v.at[p], sem_c)
        pltpu.async_copy(z_hbm.at[row_v.at[p]], zr_v.at[p], sem_c)
        pltpu.async_copy(xs_hbm.at[col_v.at[p]], xc_v.at[p], sem_c)
        pltpu.async_copy(y_hbm.at[col_v.at[p]], yc_v.at[p], sem_c)
        pltpu.async_copy(z_hbm.at[col_v.at[p]], zc_v.at[p], sem_c)

    def wait_coords(p):
        for dst in (xr_v, yr_v, zr_v, xc_v, yc_v, zc_v):
            pltpu.make_async_copy(xs_hbm.at[row_v.at[p]], dst.at[p], sem_c).wait()

    def start_coeffs(p):
        pltpu.async_copy(de0.at[didx_v.at[p]], d0_v.at[p], sem_k)
        pltpu.async_copy(de1.at[didx_v.at[p]], d1_v.at[p], sem_k)
        pltpu.async_copy(de2.at[didx_v.at[p]], d2_v.at[p], sem_k)
        pltpu.async_copy(de3.at[didx_v.at[p]], d3_v.at[p], sem_k)
        pltpu.async_copy(pa0.at[pidx_v.at[p]], p0_v.at[p], sem_k)
        pltpu.async_copy(pa1.at[pidx_v.at[p]], p1_v.at[p], sem_k)
        pltpu.async_copy(pa2.at[pidx_v.at[p]], p2_v.at[p], sem_k)
        pltpu.async_copy(pa3.at[pidx_v.at[p]], p3_v.at[p], sem_k)

    def wait_coeffs(p):
        for dst in (d0_v, d1_v, d2_v, d3_v, p0_v, p1_v, p2_v, p3_v):
            pltpu.make_async_copy(de0.at[didx_v.at[p]], dst.at[p], sem_k).wait()

    def phase_c(p):
        bcut = tblb_v[pl.ds(N_SPLINE, 16)]
        mcut = tblm_v[pl.ds(N_SPLINE, 16)]

        def pred(d2b, b, m):
            js = d2b - b
            sh = lax.shift_right_logical(m, js & 31) & 1
            return jnp.where(js < 0, False,
                             jnp.where(js > 31, True, sh == 1))

        for g in range(NG):
            sl = pl.ds(g * 16, 16)
            xsr = xr_v[p, sl]
            xsc = xc_v[p, sl]
            ti = lax.shift_right_logical(
                lax.bitcast_convert_type(xsr, jnp.int32), 31)
            tj = lax.shift_right_logical(
                lax.bitcast_convert_type(xsc, jnp.int32), 31)
            dx = _minimage(jnp.abs(xsr) - jnp.abs(xsc))
            dy = _minimage(yr_v[p, sl] - yc_v[p, sl])
            dz = _minimage(zr_v[p, sl] - zc_v[p, sl])
            d2 = ((dx * dx + dz * dz) + dy * dy) + 1e-12

            r = _approx_sqrt(d2)
            rb = jnp.minimum(jnp.maximum(r * 16665.0, 0.0), 9998.0)
            cand = rb.astype(jnp.int32)
            d2b = lax.bitcast_convert_type(d2, jnp.int32)
            b0 = plsc.load_gather(tblb_v, [cand])
            m0 = plsc.load_gather(tblm_v, [cand])
            b1 = plsc.load_gather(tblb_v, [cand + 1])
            m1 = plsc.load_gather(tblm_v, [cand + 1])
            cf = cand.astype(jnp.float32)
            cf = jnp.where(pred(d2b, b1, m1), cf + 1.0,
                           jnp.where(pred(d2b, b0, m0), cf, cf - 1.0))
            cf = jnp.minimum(jnp.maximum(cf, 0.0), 9998.0)
            idx = cf.astype(jnp.int32)
            mf = jnp.where(pred(d2b, bcut, mcut), 0.0, 1.0)

            didx_v[p, sl] = tj * (N_SPLINE - 1) + idx
            pidx_v[p, sl] = (ti * 2 + tj) * (N_SPLINE - 1) + idx
            scr_v[p, sl] = r - cf * 6.00060006000600e-05
            scr_v[p, pl.ds(ECHUNK + g * 16, 16)] = mf

    def phase_e(p):
        for g in range(NG):
            sl = pl.ds(g * 16, 16)
            dxs = scr_v[p, sl]
            mf = scr_v[p, pl.ds(ECHUNK + g * 16, 16)]
            dens = (d3_v[p, sl] + dxs * (d2_v[p, sl]
                    + dxs * (d1_v[p, sl] + dxs * d0_v[p, sl]))) * mf
            plsc.addupdate_scatter(rho_v, [row_v[p, sl]], dens)
            pv = (p3_v[p, sl] + dxs * (p2_v[p, sl]
                  + dxs * (p1_v[p, sl] + dxs * p0_v[p, sl]))) * mf
            acc_v[...] = acc_v[...] + pv

    start_rc(0, 0)
    wait_rc(0)
    start_coords(0)

    def chunk_body(k, _):
        p = k & 1

        @pl.when(k >= 1)
        def _():
            wait_coeffs(p ^ 1)
            phase_e(p ^ 1)

        @pl.when(k + 1 < NCHUNK)
        def _():
            start_rc(k + 1, p ^ 1)

        wait_coords(p)
        phase_c(p)
        start_coeffs(p)

        @pl.when(k + 1 < NCHUNK)
        def _():
            wait_rc(p ^ 1)
            start_coords(p ^ 1)
        return 0

    lax.fori_loop(0, NCHUNK, chunk_body, 0)
    lastp = (NCHUNK - 1) & 1
    wait_coeffs(lastp)
    phase_e(lastp)

    pltpu.sync_copy(rho_v, rho_out.at[wid])
    pltpu.sync_copy(acc_v, pv_out.at[wid])


_EF32 = pltpu.VMEM((2, ECHUNK), jnp.float32)
_EI32 = pltpu.VMEM((2, ECHUNK), jnp.int32)

_edge_kernel = functools.partial(
    pl.kernel,
    out_type=(jax.ShapeDtypeStruct((NW, N_ATOMS_PAD), jnp.float32),
              jax.ShapeDtypeStruct((NW, 16), jnp.float32)),
    mesh=_mesh,
    compiler_params=_cparams,
    scratch_types=[
        pltpu.VMEM((N_SPLINE + 16,), jnp.int32),
        pltpu.VMEM((N_SPLINE + 16,), jnp.int32),
        pltpu.VMEM((N_ATOMS_PAD,), jnp.float32),
        _EI32, _EI32,
        _EF32, _EF32, _EF32, _EF32, _EF32, _EF32,
        _EI32, _EI32,
        _EF32, _EF32, _EF32, _EF32, _EF32, _EF32, _EF32, _EF32,
        pltpu.VMEM((2, 2 * ECHUNK), jnp.float32),
        pltpu.VMEM((16,), jnp.float32),
        pltpu.SemaphoreType.DMA,
        pltpu.SemaphoreType.DMA,
        pltpu.SemaphoreType.DMA,
    ],
)(_edge_body)


def _atom_body(rho_parts, types_hbm, g2_hbm, em0, em1, em2, em3,
               f_out,
               g2_v, rho32_v, types_v, eidx_v, edx_v,
               e0_v, e1_v, e2_v, e3_v, facc_v, sem):
    wid = lax.axis_index("c") * NS + lax.axis_index("s")
    pltpu.sync_copy(g2_hbm, g2_v)
    facc_v[...] = jnp.zeros((16,), jnp.float32)
    lanes = lax.iota(jnp.int32, 16)

    def chunk_body(j, _):
        abase = wid * APT + j * ACHUNK
        pltpu.sync_copy(rho_parts.at[:, pl.ds(abase, ACHUNK)], rho32_v)
        pltpu.sync_copy(types_hbm.at[pl.ds(abase, ACHUNK)], types_v)

        for g in range(AG):
            sl = pl.ds(g * 16, 16)
            rho = rho32_v[0, sl]
            for p in range(1, NW):
                rho = rho + rho32_v[p, sl]

            rc = jnp.minimum(jnp.maximum(rho, -8.0), 8.0)
            sf = jnp.minimum(jnp.maximum((rc + 8.0) * 624.9375, 0.0), 9998.0)
            ei = sf.astype(jnp.int32)
            gl = plsc.load_gather(g2_v, [ei])
            gr = plsc.load_gather(g2_v, [ei + 1])
            ef = ei.astype(jnp.float32)
            ef = jnp.where(rc >= gr, ef + 1.0, jnp.where(rc < gl, ef - 1.0, ef))
            ef = jnp.minimum(jnp.maximum(ef, 0.0), 9998.0)
            eidx = ef.astype(jnp.int32)
            gsel = plsc.load_gather(g2_v, [eidx])
            eidx_v[sl] = types_v[sl] * (N_SPLINE - 1) + eidx
            edx_v[sl] = rc - gsel

        cps = []
        for q in range(ACHUNK // 112):
            qs = pl.ds(q * 112, 112)
            iq = eidx_v.at[qs]
            cps += [pltpu.async_copy(em0.at[iq], e0_v.at[qs], sem),
                    pltpu.async_copy(em1.at[iq], e1_v.at[qs], sem),
                    pltpu.async_copy(em2.at[iq], e2_v.at[qs], sem),
                    pltpu.async_copy(em3.at[iq], e3_v.at[qs], sem)]
        for c in cps:
            c.wait()

        for g in range(AG):
            sl = pl.ds(g * 16, 16)
            edx = edx_v[sl]
            fv = e3_v[sl] + edx * (e2_v[sl] + edx * (e1_v[sl] + edx * e0_v[sl]))
            aid = abase + g * 16 + lanes
            valid = jnp.where(aid < N_ATOMS, 1.0, 0.0)
            facc_v[...] = facc_v[...] + fv * valid
        return 0

    lax.fori_loop(0, NACHUNK, chunk_body, 0)
    pltpu.sync_copy(facc_v, f_out.at[wid])


_AF32 = pltpu.VMEM((ACHUNK,), jnp.float32)

_atom_kernel = functools.partial(
    pl.kernel,
    out_type=jax.ShapeDtypeStruct((NW, 16), jnp.float32),
    mesh=_mesh,
    compiler_params=_cparams,
    scratch_types=[
        pltpu.VMEM((N_SPLINE,), jnp.float32),
        pltpu.VMEM((NW, ACHUNK), jnp.float32),
        pltpu.VMEM((ACHUNK,), jnp.int32),
        pltpu.VMEM((ACHUNK,), jnp.int32),
        _AF32,
        _AF32, _AF32, _AF32, _AF32,
        pltpu.VMEM((16,), jnp.float32),
        pltpu.SemaphoreType.DMA,
    ],
)(_atom_body)


def kernel(coords, edge_index, atom_types, spline_r_x, density_coeffs,
           embed_x, embed_coeffs, pair_coeffs):
    row = edge_index[0]
    col = edge_index[1]
    xs = jnp.where(atom_types == 1, -coords[:, 0], coords[:, 0])
    y = coords[:, 1]
    z = coords[:, 2]
    de = [density_coeffs[:, k_, :].reshape(-1) for k_ in range(4)]
    pa = [pair_coeffs[:, :, k_, :].reshape(-1) for k_ in range(4)]
    em = [embed_coeffs[:, k_, :].reshape(-1) for k_ in range(4)]

    gext = jnp.concatenate([spline_r_x, jnp.full((16,), CUTOFF, jnp.float32)])
    tbl_b, tbl_m = _build_zone_tables(gext)
    types_pad = jnp.concatenate(
        [atom_types, jnp.zeros((N_ATOMS_PAD - N_ATOMS,), jnp.int32)])

    rho_parts, pv_parts = _edge_kernel(
        row, col, xs, y, z, de[0], de[1], de[2], de[3],
        pa[0], pa[1], pa[2], pa[3], tbl_b, tbl_m)
    f_parts = _atom_kernel(rho_parts, types_pad, embed_x[0],
                           em[0], em[1], em[2], em[3])
    return jnp.sum(f_parts) + 0.5 * jnp.sum(pv_parts)
